# baseline (speedup 1.0000x reference)
import functools

import jax
import jax.numpy as jnp
import numpy as np
from jax import lax
from jax.experimental import pallas as pl
from jax.experimental.pallas import tpu as pltpu
from jax.experimental.pallas import tpu_sc as plsc

B = 16384
D_E = 32
PK = 4
D_P = D_E * PK
D_M = 128
CH = 128
BLK = 2048
SCALE = np.float32(np.sqrt(float(D_M)))


def _sc_gather(r0, r1, r2, emb0p, emb1p, emb2p):
    mesh = plsc.VectorSubcoreMesh(core_axis_name="c", subcore_axis_name="s")
    nw = mesh.num_cores * mesh.num_subcores
    bpw = B // nw
    nch = bpw // CH
    rr = [r.reshape(nw, nch, CH) for r in (r0, r1, r2)]

    @functools.partial(
        pl.kernel,
        out_type=[jax.ShapeDtypeStruct((B, D_P), jnp.float32) for _ in range(3)],
        mesh=mesh,
        scratch_types=(
            [pltpu.VMEM((nch, CH), jnp.int32) for _ in range(3)]
            + [pltpu.VMEM((bpw, D_P), jnp.float32)]
            + [pltpu.SemaphoreType.DMA]
        ),
        compiler_params=pltpu.CompilerParams(use_tc_tiling_on_sc=True),
    )
    def gather_kernel(r0h, r1h, r2h, e0h, e1h, e2h, o0h, o1h, o2h,
                      i0v, i1v, i2v, rows_v, sem):
        wid = lax.axis_index("s") * mesh.num_cores + lax.axis_index("c")
        base = wid * bpw
        rhs = (r0h, r1h, r2h)
        ehs = (e0h, e1h, e2h)
        ohs = (o0h, o1h, o2h)
        ivs = (i0v, i1v, i2v)
        for t in range(3):
            pltpu.sync_copy(rhs[t].at[wid], ivs[t])
        for t in range(3):
            descs = []
            for j in range(nch):
                descs.append(pltpu.async_copy(
                    ehs[t].at[ivs[t].at[j]], rows_v.at[pl.ds(j * CH, CH)], sem))
            for dsc in descs:
                dsc.wait()
            pltpu.sync_copy(rows_v, ohs[t].at[pl.ds(base, bpw)])

    return gather_kernel(*rr, emb0p, emb1p, emb2p)


def _proj_body(x0r, x1r, x2r, s0r, s1r, s2r, f3r, lwtr, lbr,
               w0r, w1r, w2r, w3r, pbr, outr):
    x3 = f3r[...] * lwtr[...] + lbr[...]
    acc = jnp.dot(x3, w3r[...], preferred_element_type=jnp.float32)
    for xr, sr, wr in ((x0r, s0r, w0r), (x1r, s1r, w1r), (x2r, s2r, w2r)):
        xp = xr[...]
        sel = sr[...]
        xs = jnp.where(sel == 0, xp[:, 0:D_E], 0.0)
        for s in range(1, PK):
            xs += jnp.where(sel == s, xp[:, s * D_E:(s + 1) * D_E], 0.0)
        acc += jnp.dot(xs, wr[...], preferred_element_type=jnp.float32)
    outr[...] = (acc + pbr[...]) * SCALE


def kernel(f0, f1, f2, f3, emb0, emb1, emb2, lin_w, lin_b, proj_w, proj_b):
    e0p = emb0.reshape(-1, D_P)
    e1p = emb1.reshape(-1, D_P)
    e2p = emb2.reshape(-1, D_P)
    r0, r1, r2 = f0 >> 2, f1 >> 2, f2 >> 2
    s0 = (f0 & 3).astype(jnp.int32).reshape(B, 1)
    s1 = (f1 & 3).astype(jnp.int32).reshape(B, 1)
    s2 = (f2 & 3).astype(jnp.int32).reshape(B, 1)

    x0, x1, x2 = _sc_gather(r0, r1, r2, e0p, e1p, e2p)

    lin_wT = lin_w.reshape(1, 16)
    lin_b2 = lin_b.reshape(1, 16)
    w0 = proj_w[:, 0:32].T
    w1 = proj_w[:, 32:64].T
    w2 = proj_w[:, 64:96].T
    w3 = proj_w[:, 96:112].T
    pb = proj_b.reshape(1, D_M)

    cst = lambda i: (0, 0)
    row = lambda i: (i, 0)
    out = pl.pallas_call(
        _proj_body,
        grid=(B // BLK,),
        in_specs=[
            pl.BlockSpec((BLK, D_P), row),
            pl.BlockSpec((BLK, D_P), row),
            pl.BlockSpec((BLK, D_P), row),
            pl.BlockSpec((BLK, 1), row),
            pl.BlockSpec((BLK, 1), row),
            pl.BlockSpec((BLK, 1), row),
            pl.BlockSpec((BLK, 1), row),
            pl.BlockSpec((1, 16), cst),
            pl.BlockSpec((1, 16), cst),
            pl.BlockSpec((D_E, D_M), cst),
            pl.BlockSpec((D_E, D_M), cst),
            pl.BlockSpec((D_E, D_M), cst),
            pl.BlockSpec((16, D_M), cst),
            pl.BlockSpec((1, D_M), cst),
        ],
        out_specs=pl.BlockSpec((BLK, D_M), row),
        out_shape=jax.ShapeDtypeStruct((B, D_M), jnp.float32),
    )(x0, x1, x2, s0, s1, s2, f3, lin_wT, lin_b2, w0, w1, w2, w3, pb)
    return out


# speedup vs baseline: 1.0697x; 1.0697x over previous
import functools

import jax
import jax.numpy as jnp
import numpy as np
from jax import lax
from jax.experimental import pallas as pl
from jax.experimental.pallas import tpu as pltpu
from jax.experimental.pallas import tpu_sc as plsc

B = 16384
D_E = 32
D_M = 128
CH = 128
BLK = 2048
SCALE = np.float32(np.sqrt(float(D_M)))


def _sc_gather(f0, f1, f2, emb0, emb1, emb2):
    mesh = plsc.VectorSubcoreMesh(core_axis_name="c", subcore_axis_name="s")
    nw = mesh.num_cores * mesh.num_subcores
    bpw = B // nw
    nch = bpw // CH
    fr = [f.reshape(nw, nch, CH) for f in (f0, f1, f2)]

    @functools.partial(
        pl.kernel,
        out_type=[jax.ShapeDtypeStruct((B, D_E), jnp.float32) for _ in range(3)],
        mesh=mesh,
        scratch_types=(
            [pltpu.VMEM((nch, CH), jnp.int32) for _ in range(3)]
            + [pltpu.VMEM((bpw, D_E), jnp.float32) for _ in range(3)]
            + [pltpu.SemaphoreType.DMA]
        ),
        compiler_params=pltpu.CompilerParams(use_tc_tiling_on_sc=False),
    )
    def gather_kernel(f0h, f1h, f2h, e0h, e1h, e2h, o0h, o1h, o2h,
                      i0v, i1v, i2v, r0v, r1v, r2v, sem):
        wid = lax.axis_index("s") * mesh.num_cores + lax.axis_index("c")
        base = wid * bpw
        fhs = (f0h, f1h, f2h)
        ehs = (e0h, e1h, e2h)
        ohs = (o0h, o1h, o2h)
        ivs = (i0v, i1v, i2v)
        rvs = (r0v, r1v, r2v)
        for t in range(3):
            pltpu.sync_copy(fhs[t].at[wid], ivs[t])
        descs = []
        for t in range(3):
            for k in range(nch):
                descs.append(pltpu.async_copy(
                    ehs[t].at[ivs[t].at[k]], rvs[t].at[pl.ds(k * CH, CH)], sem))
        for dsc in descs:
            dsc.wait()
        for t in range(3):
            pltpu.sync_copy(rvs[t], ohs[t].at[pl.ds(base, bpw)])

    return gather_kernel(*fr, emb0, emb1, emb2)


def _proj_body(x0r, x1r, x2r, f3r, lwr, lbr, w0r, w1r, w2r, w3r, pbr, outr):
    dn = (((0,), (0,)), ((), ()))
    acc = jnp.dot(x0r[...], w0r[...], preferred_element_type=jnp.float32)
    acc += jnp.dot(x1r[...], w1r[...], preferred_element_type=jnp.float32)
    acc += jnp.dot(x2r[...], w2r[...], preferred_element_type=jnp.float32)
    v = lax.dot_general(lwr[...], w3r[...], dn,
                        preferred_element_type=jnp.float32)
    cb = jnp.dot(lbr[...], w3r[...],
                 preferred_element_type=jnp.float32)
    acc += jnp.dot(f3r[...], v, preferred_element_type=jnp.float32)
    outr[...] = (acc + cb + pbr[...]) * SCALE


def kernel(f0, f1, f2, f3, emb0, emb1, emb2, lin_w, lin_b, proj_w, proj_b):
    x0, x1, x2 = _sc_gather(f0, f1, f2, emb0, emb1, emb2)

    w0 = proj_w[:, 0:32].T
    w1 = proj_w[:, 32:64].T
    w2 = proj_w[:, 64:96].T
    w3 = proj_w[:, 96:112].T
    pb = proj_b.reshape(1, D_M)
    lb = lin_b.reshape(1, 16)

    cst = lambda i: (0, 0)
    row = lambda i: (i, 0)
    out = pl.pallas_call(
        _proj_body,
        grid=(B // BLK,),
        in_specs=[
            pl.BlockSpec((BLK, D_E), row),
            pl.BlockSpec((BLK, D_E), row),
            pl.BlockSpec((BLK, D_E), row),
            pl.BlockSpec((BLK, 1), row),
            pl.BlockSpec((16, 1), cst),
            pl.BlockSpec((1, 16), cst),
            pl.BlockSpec((D_E, D_M), cst),
            pl.BlockSpec((D_E, D_M), cst),
            pl.BlockSpec((D_E, D_M), cst),
            pl.BlockSpec((16, D_M), cst),
            pl.BlockSpec((1, D_M), cst),
        ],
        out_specs=pl.BlockSpec((BLK, D_M), row),
        out_shape=jax.ShapeDtypeStruct((B, D_M), jnp.float32),
    )(x0, x1, x2, f3, lin_w, lb, w0, w1, w2, w3, pb)
    return out


# speedup vs baseline: 2.2623x; 2.1150x over previous
import functools

import jax
import jax.numpy as jnp
import numpy as np
from jax import lax
from jax.experimental import pallas as pl
from jax.experimental.pallas import tpu as pltpu
from jax.experimental.pallas import tpu_sc as plsc

B = 16384
D_E = 32
D_M = 128
G = 16
CH = 128
BLK = 2048
SCALE = np.float32(np.sqrt(float(D_M)))


def _sc_gather_big(f0, e0t):
    mesh = plsc.VectorSubcoreMesh(core_axis_name="c", subcore_axis_name="s")
    nw = mesh.num_cores * mesh.num_subcores
    bpw = B // nw
    ngr = bpw // G
    fr = f0.reshape(nw, bpw)

    @functools.partial(
        pl.kernel,
        out_type=jax.ShapeDtypeStruct((B, D_E), jnp.float32),
        mesh=mesh,
        scratch_types=(
            [pltpu.VMEM((bpw,), jnp.int32),
             pltpu.VMEM((G // 2, D_E, 128), jnp.float32),
             pltpu.VMEM((bpw, D_E), jnp.float32),
             pltpu.SemaphoreType.DMA]
        ),
        compiler_params=pltpu.CompilerParams(use_tc_tiling_on_sc=True,
                                             needs_layout_passes=False),
    )
    def gather_kernel(fh, eh, oh, idx_v, gb, rows_v, sem):
        wid = lax.axis_index("s") * mesh.num_cores + lax.axis_index("c")
        base = wid * bpw
        feats = lax.iota(jnp.int32, 16)
        pltpu.sync_copy(fh.at[wid], idx_v)

        def group(g, _):
            rv = idx_v[pl.ds(g * G, G)]
            r_al_v = lax.bitwise_and(rv, jnp.int32(-128))
            lane_v = lax.bitwise_and(rv, jnp.int32(127))
            for w in range(2):
                descs = []
                for j in range(G // 2):
                    r_al = pl.multiple_of(r_al_v[w * (G // 2) + j], 128)
                    descs.append(pltpu.async_copy(
                        eh.at[:, pl.ds(r_al, 128)], gb.at[j], sem))
                for dsc in descs:
                    dsc.wait()
                for j in range(G // 2):
                    i = g * G + w * (G // 2) + j
                    bj = jnp.full((16,), j, jnp.int32)
                    bl = jnp.full((16,), lane_v[w * (G // 2) + j], jnp.int32)
                    lo = plsc.load_gather(gb, [bj, feats, bl])
                    hi = plsc.load_gather(gb, [bj, feats + 16, bl])
                    rows_v[i, pl.ds(0, 16)] = lo
                    rows_v[i, pl.ds(16, 16)] = hi
            return 0

        lax.fori_loop(0, ngr, group, 0)
        pltpu.sync_copy(rows_v, oh.at[pl.ds(base, bpw)])

    return gather_kernel(fr, e0t)


def _sc_gather_small(f1, f2, emb1, emb2):
    mesh = plsc.VectorSubcoreMesh(core_axis_name="c", subcore_axis_name="s")
    nw = mesh.num_cores * mesh.num_subcores
    bpw = B // nw
    nch = bpw // CH
    fr = [f.reshape(nw, nch, CH) for f in (f1, f2)]

    @functools.partial(
        pl.kernel,
        out_type=[jax.ShapeDtypeStruct((B, D_E), jnp.float32) for _ in range(2)],
        mesh=mesh,
        scratch_types=(
            [pltpu.VMEM((nch, CH), jnp.int32) for _ in range(2)]
            + [pltpu.VMEM((bpw, D_E), jnp.float32) for _ in range(2)]
            + [pltpu.SemaphoreType.DMA]
        ),
        compiler_params=pltpu.CompilerParams(use_tc_tiling_on_sc=False),
    )
    def gather_kernel(f1h, f2h, e1h, e2h, o1h, o2h,
                      i1v, i2v, r1v, r2v, sem):
        wid = lax.axis_index("s") * mesh.num_cores + lax.axis_index("c")
        base = wid * bpw
        fhs = (f1h, f2h)
        ehs = (e1h, e2h)
        ohs = (o1h, o2h)
        ivs = (i1v, i2v)
        rvs = (r1v, r2v)
        for t in range(2):
            pltpu.sync_copy(fhs[t].at[wid], ivs[t])
        descs = []
        for t in range(2):
            for k in range(nch):
                descs.append(pltpu.async_copy(
                    ehs[t].at[ivs[t].at[k]], rvs[t].at[pl.ds(k * CH, CH)], sem))
        for dsc in descs:
            dsc.wait()
        for t in range(2):
            pltpu.sync_copy(rvs[t], ohs[t].at[pl.ds(base, bpw)])

    return gather_kernel(*fr, emb1, emb2)


def _proj_body(x0r, x1r, x2r, f3r, lwr, lbr, w0r, w1r, w2r, w3r, pbr, outr):
    dn = (((0,), (0,)), ((), ()))
    acc = jnp.dot(x0r[...], w0r[...], preferred_element_type=jnp.float32)
    acc += jnp.dot(x1r[...], w1r[...], preferred_element_type=jnp.float32)
    acc += jnp.dot(x2r[...], w2r[...], preferred_element_type=jnp.float32)
    v = lax.dot_general(lwr[...], w3r[...], dn,
                        preferred_element_type=jnp.float32)
    cb = jnp.dot(lbr[...], w3r[...],
                 preferred_element_type=jnp.float32)
    acc += jnp.dot(f3r[...], v, preferred_element_type=jnp.float32)
    outr[...] = (acc + cb + pbr[...]) * SCALE


def kernel(f0, f1, f2, f3, emb0, emb1, emb2, lin_w, lin_b, proj_w, proj_b):
    x0 = _sc_gather_big(f0, emb0.T)
    x1, x2 = _sc_gather_small(f1, f2, emb1, emb2)

    w0 = proj_w[:, 0:32].T
    w1 = proj_w[:, 32:64].T
    w2 = proj_w[:, 64:96].T
    w3 = proj_w[:, 96:112].T
    pb = proj_b.reshape(1, D_M)
    lb = lin_b.reshape(1, 16)

    cst = lambda i: (0, 0)
    row = lambda i: (i, 0)
    out = pl.pallas_call(
        _proj_body,
        grid=(B // BLK,),
        in_specs=[
            pl.BlockSpec((BLK, D_E), row),
            pl.BlockSpec((BLK, D_E), row),
            pl.BlockSpec((BLK, D_E), row),
            pl.BlockSpec((BLK, 1), row),
            pl.BlockSpec((16, 1), cst),
            pl.BlockSpec((1, 16), cst),
            pl.BlockSpec((D_E, D_M), cst),
            pl.BlockSpec((D_E, D_M), cst),
            pl.BlockSpec((D_E, D_M), cst),
            pl.BlockSpec((16, D_M), cst),
            pl.BlockSpec((1, D_M), cst),
        ],
        out_specs=pl.BlockSpec((BLK, D_M), row),
        out_shape=jax.ShapeDtypeStruct((B, D_M), jnp.float32),
    )(x0, x1, x2, f3, lin_w, lb, w0, w1, w2, w3, pb)
    return out


# speedup vs baseline: 2.5273x; 1.1171x over previous
import functools

import jax
import jax.numpy as jnp
import numpy as np
from jax import lax
from jax.experimental import pallas as pl
from jax.experimental.pallas import tpu as pltpu
from jax.experimental.pallas import tpu_sc as plsc

B = 16384
D_E = 32
D_M = 128
G = 16
CH = 128
BLK = 2048
SCALE = np.float32(np.sqrt(float(D_M)))


def _sc_gather_big(f0, e0t):
    mesh = plsc.VectorSubcoreMesh(core_axis_name="c", subcore_axis_name="s")
    nw = mesh.num_cores * mesh.num_subcores
    bpw = B // nw
    ngr = bpw // G
    fr = f0.reshape(nw, bpw)

    @functools.partial(
        pl.kernel,
        out_type=jax.ShapeDtypeStruct((B, D_E), jnp.float32),
        mesh=mesh,
        scratch_types=(
            [pltpu.VMEM((bpw,), jnp.int32),
             pltpu.VMEM((G // 2, D_E, 128), jnp.float32),
             pltpu.VMEM((bpw, D_E), jnp.float32),
             pltpu.SemaphoreType.DMA]
        ),
        compiler_params=pltpu.CompilerParams(use_tc_tiling_on_sc=True,
                                             needs_layout_passes=False),
    )
    def gather_kernel(fh, eh, oh, idx_v, gb, rows_v, sem):
        wid = lax.axis_index("s") * mesh.num_cores + lax.axis_index("c")
        base = wid * bpw
        feats = lax.iota(jnp.int32, 16)
        pltpu.sync_copy(fh.at[wid], idx_v)

        def group(g, _):
            rv = idx_v[pl.ds(g * G, G)]
            r_al_v = lax.bitwise_and(rv, jnp.int32(-128))
            lane_v = lax.bitwise_and(rv, jnp.int32(127))
            for w in range(2):
                descs = []
                for j in range(G // 2):
                    r_al = pl.multiple_of(r_al_v[w * (G // 2) + j], 128)
                    descs.append(pltpu.async_copy(
                        eh.at[:, pl.ds(r_al, 128)], gb.at[j], sem))
                for dsc in descs:
                    dsc.wait()
                for j in range(G // 2):
                    i = g * G + w * (G // 2) + j
                    bj = jnp.full((16,), j, jnp.int32)
                    bl = jnp.full((16,), lane_v[w * (G // 2) + j], jnp.int32)
                    lo = plsc.load_gather(gb, [bj, feats, bl])
                    hi = plsc.load_gather(gb, [bj, feats + 16, bl])
                    rows_v[i, pl.ds(0, 16)] = lo
                    rows_v[i, pl.ds(16, 16)] = hi
            return 0

        lax.fori_loop(0, ngr, group, 0)
        pltpu.sync_copy(rows_v, oh.at[pl.ds(base, bpw)])

    return gather_kernel(fr, e0t)


def _sc_gather_small(f1, f2, emb1, emb2):
    mesh = plsc.VectorSubcoreMesh(core_axis_name="c", subcore_axis_name="s")
    nw = mesh.num_cores * mesh.num_subcores
    bpw = B // nw
    nch = bpw // CH
    fr = [f.reshape(nw, nch, CH) for f in (f1, f2)]

    @functools.partial(
        pl.kernel,
        out_type=[jax.ShapeDtypeStruct((B, D_E), jnp.float32) for _ in range(2)],
        mesh=mesh,
        scratch_types=(
            [pltpu.VMEM((nch, CH), jnp.int32) for _ in range(2)]
            + [pltpu.VMEM((bpw, D_E), jnp.float32) for _ in range(2)]
            + [pltpu.SemaphoreType.DMA]
        ),
        compiler_params=pltpu.CompilerParams(use_tc_tiling_on_sc=False),
    )
    def gather_kernel(f1h, f2h, e1h, e2h, o1h, o2h,
                      i1v, i2v, r1v, r2v, sem):
        wid = lax.axis_index("s") * mesh.num_cores + lax.axis_index("c")
        base = wid * bpw
        fhs = (f1h, f2h)
        ehs = (e1h, e2h)
        ohs = (o1h, o2h)
        ivs = (i1v, i2v)
        rvs = (r1v, r2v)
        for t in range(2):
            pltpu.sync_copy(fhs[t].at[wid], ivs[t])
        descs = []
        for t in range(2):
            for k in range(nch):
                descs.append(pltpu.async_copy(
                    ehs[t].at[ivs[t].at[k]], rvs[t].at[pl.ds(k * CH, CH)], sem))
        for dsc in descs:
            dsc.wait()
        for t in range(2):
            pltpu.sync_copy(rvs[t], ohs[t].at[pl.ds(base, bpw)])

    return gather_kernel(*fr, emb1, emb2)


def _proj_body(x0r, x1r, x2r, f3r, lwr, lbr, w0r, w1r, w2r, w3r, pbr, outr):
    dn = (((0,), (0,)), ((), ()))
    acc = jnp.dot(x0r[...], w0r[...], preferred_element_type=jnp.float32)
    acc += jnp.dot(x1r[...], w1r[...], preferred_element_type=jnp.float32)
    acc += jnp.dot(x2r[...], w2r[...], preferred_element_type=jnp.float32)
    v = lax.dot_general(lwr[...], w3r[...], dn,
                        preferred_element_type=jnp.float32)
    cb = jnp.dot(lbr[...], w3r[...],
                 preferred_element_type=jnp.float32)
    acc += jnp.dot(f3r[...], v, preferred_element_type=jnp.float32)
    outr[...] = (acc + cb + pbr[...]) * SCALE


def kernel(f0, f1, f2, f3, emb0, emb1, emb2, lin_w, lin_b, proj_w, proj_b):
    x0 = _sc_gather_big(f0, emb0.T)
    dep = (x0[0, 0] * 0.0).astype(jnp.int32)
    x1, x2 = _sc_gather_small(f1 + dep, f2 + dep, emb1, emb2)

    w0 = proj_w[:, 0:32].T
    w1 = proj_w[:, 32:64].T
    w2 = proj_w[:, 64:96].T
    w3 = proj_w[:, 96:112].T
    pb = proj_b.reshape(1, D_M)
    lb = lin_b.reshape(1, 16)

    cst = lambda i: (0, 0)
    row = lambda i: (i, 0)
    out = pl.pallas_call(
        _proj_body,
        grid=(B // BLK,),
        in_specs=[
            pl.BlockSpec((BLK, D_E), row),
            pl.BlockSpec((BLK, D_E), row),
            pl.BlockSpec((BLK, D_E), row),
            pl.BlockSpec((BLK, 1), row),
            pl.BlockSpec((16, 1), cst),
            pl.BlockSpec((1, 16), cst),
            pl.BlockSpec((D_E, D_M), cst),
            pl.BlockSpec((D_E, D_M), cst),
            pl.BlockSpec((D_E, D_M), cst),
            pl.BlockSpec((16, D_M), cst),
            pl.BlockSpec((1, D_M), cst),
        ],
        out_specs=pl.BlockSpec((BLK, D_M), row),
        out_shape=jax.ShapeDtypeStruct((B, D_M), jnp.float32),
    )(x0, x1, x2, f3, lin_w, lb, w0, w1, w2, w3, pb)
    return out


# speedup vs baseline: 2.6802x; 1.0605x over previous
import functools

import jax
import jax.numpy as jnp
import numpy as np
from jax import lax
from jax.experimental import pallas as pl
from jax.experimental.pallas import tpu as pltpu
from jax.experimental.pallas import tpu_sc as plsc

B = 16384
D_E = 32
D_M = 128
G = 16
CH = 128
BLK = 2048
SCALE = np.float32(np.sqrt(float(D_M)))


def _sc_gather_big(f0, e0t):
    mesh = plsc.VectorSubcoreMesh(core_axis_name="c", subcore_axis_name="s")
    nw = mesh.num_cores * mesh.num_subcores
    bpw = B // nw
    ngr = bpw // G
    fr = f0.reshape(nw, bpw)

    @functools.partial(
        pl.kernel,
        out_type=jax.ShapeDtypeStruct((B, D_E), jnp.float32),
        mesh=mesh,
        scratch_types=(
            [pltpu.VMEM((bpw,), jnp.int32),
             pltpu.VMEM((2, G // 4, D_E, 128), jnp.float32),
             pltpu.VMEM((bpw, D_E), jnp.float32),
             pltpu.SemaphoreType.DMA,
             pltpu.SemaphoreType.DMA]
        ),
        compiler_params=pltpu.CompilerParams(use_tc_tiling_on_sc=True,
                                             needs_layout_passes=False),
    )
    def gather_kernel(fh, eh, oh, idx_v, gb, rows_v, sem0, sem1):
        wid = lax.axis_index("s") * mesh.num_cores + lax.axis_index("c")
        base = wid * bpw
        feats = lax.iota(jnp.int32, 16)
        pltpu.sync_copy(fh.at[wid], idx_v)

        W = G // 4

        def fire_q(rv, q, buf):
            r_al_v = lax.bitwise_and(rv, jnp.int32(-128))
            for j in range(W):
                r_al = pl.multiple_of(r_al_v[q * W + j], 128)
                pltpu.async_copy(
                    eh.at[:, pl.ds(r_al, 128)], gb.at[buf, j],
                    sem0 if buf == 0 else sem1)

        def drain_extract(rv, q, buf, gbase):
            lane_v = lax.bitwise_and(rv, jnp.int32(127))
            for j in range(W):
                pltpu.make_async_copy(
                    eh.at[:, pl.ds(0, 128)], gb.at[buf, j],
                    sem0 if buf == 0 else sem1).wait()
            for j in range(W):
                bj = jnp.full((16,), j, jnp.int32)
                bl = jnp.full((16,), lane_v[q * W + j], jnp.int32)
                lo = plsc.load_gather(gb.at[buf], [bj, feats, bl])
                hi = plsc.load_gather(gb.at[buf], [bj, feats + 16, bl])
                rows_v[gbase + q * W + j, pl.ds(0, 16)] = lo
                rows_v[gbase + q * W + j, pl.ds(16, 16)] = hi

        rv0 = idx_v[pl.ds(0, G)]
        fire_q(rv0, 0, 0)

        def group(g, _):
            rv = idx_v[pl.ds(g * G, G)]
            gbase = g * G
            fire_q(rv, 1, 1)
            drain_extract(rv, 0, 0, gbase)
            fire_q(rv, 2, 0)
            drain_extract(rv, 1, 1, gbase)
            fire_q(rv, 3, 1)
            drain_extract(rv, 2, 0, gbase)

            @pl.when(g + 1 < ngr)
            def _():
                rvn = idx_v[pl.ds((g + 1) * G, G)]
                fire_q(rvn, 0, 0)

            drain_extract(rv, 3, 1, gbase)
            return 0

        lax.fori_loop(0, ngr, group, 0)
        pltpu.sync_copy(rows_v, oh.at[pl.ds(base, bpw)])

    return gather_kernel(fr, e0t)


def _sc_gather_small(f1, f2, emb1, emb2):
    mesh = plsc.VectorSubcoreMesh(core_axis_name="c", subcore_axis_name="s")
    nw = mesh.num_cores * mesh.num_subcores
    bpw = B // nw
    nch = bpw // CH
    fr = [f.reshape(nw, nch, CH) for f in (f1, f2)]

    @functools.partial(
        pl.kernel,
        out_type=[jax.ShapeDtypeStruct((B, D_E), jnp.float32) for _ in range(2)],
        mesh=mesh,
        scratch_types=(
            [pltpu.VMEM((nch, CH), jnp.int32) for _ in range(2)]
            + [pltpu.VMEM((bpw, D_E), jnp.float32) for _ in range(2)]
            + [pltpu.SemaphoreType.DMA]
        ),
        compiler_params=pltpu.CompilerParams(use_tc_tiling_on_sc=False),
    )
    def gather_kernel(f1h, f2h, e1h, e2h, o1h, o2h,
                      i1v, i2v, r1v, r2v, sem):
        wid = lax.axis_index("s") * mesh.num_cores + lax.axis_index("c")
        base = wid * bpw
        fhs = (f1h, f2h)
        ehs = (e1h, e2h)
        ohs = (o1h, o2h)
        ivs = (i1v, i2v)
        rvs = (r1v, r2v)
        for t in range(2):
            pltpu.sync_copy(fhs[t].at[wid], ivs[t])
        descs = []
        for t in range(2):
            for k in range(nch):
                descs.append(pltpu.async_copy(
                    ehs[t].at[ivs[t].at[k]], rvs[t].at[pl.ds(k * CH, CH)], sem))
        for dsc in descs:
            dsc.wait()
        for t in range(2):
            pltpu.sync_copy(rvs[t], ohs[t].at[pl.ds(base, bpw)])

    return gather_kernel(*fr, emb1, emb2)


def _proj_body(x0r, x1r, x2r, f3r, lwr, lbr, w0r, w1r, w2r, w3r, pbr, outr):
    dn = (((0,), (0,)), ((), ()))
    acc = jnp.dot(x0r[...], w0r[...], preferred_element_type=jnp.float32)
    acc += jnp.dot(x1r[...], w1r[...], preferred_element_type=jnp.float32)
    acc += jnp.dot(x2r[...], w2r[...], preferred_element_type=jnp.float32)
    v = lax.dot_general(lwr[...], w3r[...], dn,
                        preferred_element_type=jnp.float32)
    cb = jnp.dot(lbr[...], w3r[...],
                 preferred_element_type=jnp.float32)
    acc += jnp.dot(f3r[...], v, preferred_element_type=jnp.float32)
    outr[...] = (acc + cb + pbr[...]) * SCALE


def kernel(f0, f1, f2, f3, emb0, emb1, emb2, lin_w, lin_b, proj_w, proj_b):
    x0 = _sc_gather_big(f0, emb0.T)
    dep = (x0[0, 0] * 0.0).astype(jnp.int32)
    x1, x2 = _sc_gather_small(f1 + dep, f2 + dep, emb1, emb2)

    w0 = proj_w[:, 0:32].T
    w1 = proj_w[:, 32:64].T
    w2 = proj_w[:, 64:96].T
    w3 = proj_w[:, 96:112].T
    pb = proj_b.reshape(1, D_M)
    lb = lin_b.reshape(1, 16)

    cst = lambda i: (0, 0)
    row = lambda i: (i, 0)
    out = pl.pallas_call(
        _proj_body,
        grid=(B // BLK,),
        in_specs=[
            pl.BlockSpec((BLK, D_E), row),
            pl.BlockSpec((BLK, D_E), row),
            pl.BlockSpec((BLK, D_E), row),
            pl.BlockSpec((BLK, 1), row),
            pl.BlockSpec((16, 1), cst),
            pl.BlockSpec((1, 16), cst),
            pl.BlockSpec((D_E, D_M), cst),
            pl.BlockSpec((D_E, D_M), cst),
            pl.BlockSpec((D_E, D_M), cst),
            pl.BlockSpec((16, D_M), cst),
            pl.BlockSpec((1, D_M), cst),
        ],
        out_specs=pl.BlockSpec((BLK, D_M), row),
        out_shape=jax.ShapeDtypeStruct((B, D_M), jnp.float32),
    )(x0, x1, x2, f3, lin_w, lb, w0, w1, w2, w3, pb)
    return out


# speedup vs baseline: 2.7050x; 1.0092x over previous
import functools

import jax
import jax.numpy as jnp
import numpy as np
from jax import lax
from jax.experimental import pallas as pl
from jax.experimental.pallas import tpu as pltpu
from jax.experimental.pallas import tpu_sc as plsc

B = 16384
D_E = 32
D_M = 128
G = 16
CH = 128
BLK = 4096
SCALE = np.float32(np.sqrt(float(D_M)))


def _sc_gather_big(f0, e0t):
    mesh = plsc.VectorSubcoreMesh(core_axis_name="c", subcore_axis_name="s")
    nw = mesh.num_cores * mesh.num_subcores
    bpw = B // nw
    ngr = bpw // G
    fr = f0.reshape(nw, bpw)

    @functools.partial(
        pl.kernel,
        out_type=jax.ShapeDtypeStruct((B, D_E), jnp.float32),
        mesh=mesh,
        scratch_types=(
            [pltpu.VMEM((bpw,), jnp.int32),
             pltpu.VMEM((2, G // 4, D_E, 128), jnp.float32),
             pltpu.VMEM((bpw, D_E), jnp.float32),
             pltpu.SemaphoreType.DMA,
             pltpu.SemaphoreType.DMA]
        ),
        compiler_params=pltpu.CompilerParams(use_tc_tiling_on_sc=True,
                                             needs_layout_passes=False),
    )
    def gather_kernel(fh, eh, oh, idx_v, gb, rows_v, sem0, sem1):
        wid = lax.axis_index("s") * mesh.num_cores + lax.axis_index("c")
        base = wid * bpw
        feats = lax.iota(jnp.int32, 16)
        pltpu.sync_copy(fh.at[wid], idx_v)

        W = G // 4

        def fire_q(rv, q, buf):
            r_al_v = lax.bitwise_and(rv, jnp.int32(-128))
            for j in range(W):
                r_al = pl.multiple_of(r_al_v[q * W + j], 128)
                pltpu.async_copy(
                    eh.at[:, pl.ds(r_al, 128)], gb.at[buf, j],
                    sem0 if buf == 0 else sem1)

        def drain_extract(rv, q, buf, gbase):
            lane_v = lax.bitwise_and(rv, jnp.int32(127))
            for j in range(W):
                pltpu.make_async_copy(
                    eh.at[:, pl.ds(0, 128)], gb.at[buf, j],
                    sem0 if buf == 0 else sem1).wait()
            for j in range(W):
                bj = jnp.full((16,), j, jnp.int32)
                bl = jnp.full((16,), lane_v[q * W + j], jnp.int32)
                lo = plsc.load_gather(gb.at[buf], [bj, feats, bl])
                hi = plsc.load_gather(gb.at[buf], [bj, feats + 16, bl])
                rows_v[gbase + q * W + j, pl.ds(0, 16)] = lo
                rows_v[gbase + q * W + j, pl.ds(16, 16)] = hi

        rv0 = idx_v[pl.ds(0, G)]
        fire_q(rv0, 0, 0)

        def group(g, _):
            rv = idx_v[pl.ds(g * G, G)]
            gbase = g * G
            fire_q(rv, 1, 1)
            drain_extract(rv, 0, 0, gbase)
            fire_q(rv, 2, 0)
            drain_extract(rv, 1, 1, gbase)
            fire_q(rv, 3, 1)
            drain_extract(rv, 2, 0, gbase)

            @pl.when(g + 1 < ngr)
            def _():
                rvn = idx_v[pl.ds((g + 1) * G, G)]
                fire_q(rvn, 0, 0)

            drain_extract(rv, 3, 1, gbase)
            return 0

        lax.fori_loop(0, ngr, group, 0)
        pltpu.sync_copy(rows_v, oh.at[pl.ds(base, bpw)])

    return gather_kernel(fr, e0t)


def _sc_gather_small(f1, f2, emb1, emb2):
    mesh = plsc.VectorSubcoreMesh(core_axis_name="c", subcore_axis_name="s")
    nw = mesh.num_cores * mesh.num_subcores
    bpw = B // nw
    nch = bpw // CH
    fr = [f.reshape(nw, nch, CH) for f in (f1, f2)]

    @functools.partial(
        pl.kernel,
        out_type=[jax.ShapeDtypeStruct((B, D_E), jnp.float32) for _ in range(2)],
        mesh=mesh,
        scratch_types=(
            [pltpu.VMEM((nch, CH), jnp.int32) for _ in range(2)]
            + [pltpu.VMEM((bpw, D_E), jnp.float32) for _ in range(2)]
            + [pltpu.SemaphoreType.DMA]
        ),
        compiler_params=pltpu.CompilerParams(use_tc_tiling_on_sc=False),
    )
    def gather_kernel(f1h, f2h, e1h, e2h, o1h, o2h,
                      i1v, i2v, r1v, r2v, sem):
        wid = lax.axis_index("s") * mesh.num_cores + lax.axis_index("c")
        base = wid * bpw
        fhs = (f1h, f2h)
        ehs = (e1h, e2h)
        ohs = (o1h, o2h)
        ivs = (i1v, i2v)
        rvs = (r1v, r2v)
        for t in range(2):
            pltpu.sync_copy(fhs[t].at[wid], ivs[t])
        descs = []
        for t in range(2):
            for k in range(nch):
                descs.append(pltpu.async_copy(
                    ehs[t].at[ivs[t].at[k]], rvs[t].at[pl.ds(k * CH, CH)], sem))
        for dsc in descs:
            dsc.wait()
        for t in range(2):
            pltpu.sync_copy(rvs[t], ohs[t].at[pl.ds(base, bpw)])

    return gather_kernel(*fr, emb1, emb2)


def _proj_body(x0r, x1r, x2r, f3r, lwr, lbr, w0r, w1r, w2r, w3r, pbr, outr):
    dn = (((0,), (0,)), ((), ()))
    acc = jnp.dot(x0r[...], w0r[...], preferred_element_type=jnp.float32)
    acc += jnp.dot(x1r[...], w1r[...], preferred_element_type=jnp.float32)
    acc += jnp.dot(x2r[...], w2r[...], preferred_element_type=jnp.float32)
    v = lax.dot_general(lwr[...], w3r[...], dn,
                        preferred_element_type=jnp.float32)
    cb = jnp.dot(lbr[...], w3r[...],
                 preferred_element_type=jnp.float32)
    acc += jnp.dot(f3r[...], v, preferred_element_type=jnp.float32)
    outr[...] = (acc + cb + pbr[...]) * SCALE


def kernel(f0, f1, f2, f3, emb0, emb1, emb2, lin_w, lin_b, proj_w, proj_b):
    x0 = _sc_gather_big(f0, emb0.T)
    dep = (x0[0, 0] * 0.0).astype(jnp.int32)
    x1, x2 = _sc_gather_small(f1 + dep, f2 + dep, emb1, emb2)

    w0 = proj_w[:, 0:32].T
    w1 = proj_w[:, 32:64].T
    w2 = proj_w[:, 64:96].T
    w3 = proj_w[:, 96:112].T
    pb = proj_b.reshape(1, D_M)
    lb = lin_b.reshape(1, 16)

    cst = lambda i: (0, 0)
    row = lambda i: (i, 0)
    out = pl.pallas_call(
        _proj_body,
        grid=(B // BLK,),
        in_specs=[
            pl.BlockSpec((BLK, D_E), row),
            pl.BlockSpec((BLK, D_E), row),
            pl.BlockSpec((BLK, D_E), row),
            pl.BlockSpec((BLK, 1), row),
            pl.BlockSpec((16, 1), cst),
            pl.BlockSpec((1, 16), cst),
            pl.BlockSpec((D_E, D_M), cst),
            pl.BlockSpec((D_E, D_M), cst),
            pl.BlockSpec((D_E, D_M), cst),
            pl.BlockSpec((16, D_M), cst),
            pl.BlockSpec((1, D_M), cst),
        ],
        out_specs=pl.BlockSpec((BLK, D_M), row),
        out_shape=jax.ShapeDtypeStruct((B, D_M), jnp.float32),
    )(x0, x1, x2, f3, lin_w, lb, w0, w1, w2, w3, pb)
    return out
